# baseline (speedup 1.0000x reference)
import functools
import math

import jax
import jax.numpy as jnp
from jax.experimental import pallas as pl

_B = 16
_N = 2048
_D = 3
_C = 128
_M = 1000


def _gelu(t):
    return t * (0.5 + 0.5 * jax.lax.erf(t * (1.0 / math.sqrt(2.0))))


_BPS = 4
_NCHUNK = _B // _BPS


def _body(x_ref, we1_ref, be1_ref, we2_ref, be2_ref, we3_ref, be3_ref,
          wd1_ref, bd1_ref, wd2_ref, bd2_ref, wd3_ref, bd3_ref,
          z_ref, o_ref):
    zs = []
    for c in range(_NCHUNK):
        xb = x_ref[pl.ds(c * _BPS * _N, _BPS * _N), :]
        h = (jnp.dot(xb, we1_ref[...], preferred_element_type=jnp.float32)
             + be1_ref[...])
        h = _gelu(h)
        h = _gelu(jnp.dot(h, we2_ref[...], preferred_element_type=jnp.float32)
                  + be2_ref[...])
        a = (jnp.dot(h, we3_ref[...], preferred_element_type=jnp.float32)
             + be3_ref[...])
        a3 = a.reshape(_BPS, _N, _C)
        zs.append(jnp.maximum(_gelu(jnp.max(a3, axis=1)),
                              _gelu(jnp.min(a3, axis=1))))
    z = jnp.concatenate(zs, axis=0)
    z_ref[...] = z
    d = _gelu(jnp.dot(z, wd1_ref[...],
                      preferred_element_type=jnp.float32) + bd1_ref[...])
    d = _gelu(jnp.dot(d, wd2_ref[...],
                      preferred_element_type=jnp.float32) + bd2_ref[...])
    o_ref[...] = (jnp.dot(d, wd3_ref[...],
                          preferred_element_type=jnp.float32) + bd3_ref[...])


@functools.partial(jax.jit, static_argnames=())
def kernel(x, We1, be1, We2, be2, We3, be3, Wd1, bd1, Wd2, bd2, Wd3, bd3):
    full = lambda s: pl.BlockSpec(s, lambda: (0,) * len(s))
    z, out = pl.pallas_call(
        _body,
        in_specs=[
            full((_B * _N, _D)),
            full((_D, _C)), full((1, _C)),
            full((_C, _C)), full((1, _C)),
            full((_C, _C)), full((1, _C)),
            full((_C, _C)), full((1, _C)),
            full((_C, _C)), full((1, _C)),
            full((_C, _M * _D)), full((1, _M * _D)),
        ],
        out_specs=[
            full((_B, _C)),
            full((_B, _M * _D)),
        ],
        out_shape=[
            jax.ShapeDtypeStruct((_B, _C), jnp.float32),
            jax.ShapeDtypeStruct((_B, _M * _D), jnp.float32),
        ],
    )(x.reshape(_B * _N, _D), We1, be1.reshape(1, _C), We2, be2.reshape(1, _C),
      We3, be3.reshape(1, _C), Wd1, bd1.reshape(1, _C),
      Wd2, bd2.reshape(1, _C), Wd3, bd3.reshape(1, _M * _D))
    return (out.reshape(_B, _M, _D), z)


# speedup vs baseline: 1.0388x; 1.0388x over previous
import functools
import math

import jax
import jax.numpy as jnp
from jax.experimental import pallas as pl

_B = 16
_N = 2048
_D = 3
_C = 128
_M = 1000


def _gelu(t):
    return t * (0.5 + 0.5 * jax.lax.erf(t * (1.0 / math.sqrt(2.0))))


_BPS = 2
_NSTEPS = _B // _BPS


def _body(x_ref, we1_ref, be1_ref, we2_ref, be2_ref, we3_ref, be3_ref,
          wd1_ref, bd1_ref, wd2_ref, bd2_ref, wd3_ref, bd3_ref,
          z_ref, o_ref):
    b = pl.program_id(0)

    @pl.when(b < _NSTEPS)
    def _encode():
        xb = x_ref[...]
        h = (jnp.dot(xb, we1_ref[...], preferred_element_type=jnp.float32)
             + be1_ref[...])
        h = _gelu(h)
        h = _gelu(jnp.dot(h, we2_ref[...], preferred_element_type=jnp.float32)
                  + be2_ref[...])
        a = (jnp.dot(h, we3_ref[...], preferred_element_type=jnp.float32)
             + be3_ref[...])
        a3 = a.reshape(_BPS, _N, _C)
        zrow = jnp.maximum(_gelu(jnp.max(a3, axis=1)),
                           _gelu(jnp.min(a3, axis=1)))
        z_ref[pl.ds(b * _BPS, _BPS), :] = zrow

    @pl.when(b == _NSTEPS)
    def _decode():
        d = _gelu(jnp.dot(z_ref[...], wd1_ref[...],
                          preferred_element_type=jnp.float32) + bd1_ref[...])
        d = _gelu(jnp.dot(d, wd2_ref[...],
                          preferred_element_type=jnp.float32) + bd2_ref[...])
        o_ref[...] = (jnp.dot(d, wd3_ref[...],
                              preferred_element_type=jnp.float32)
                      + bd3_ref[...])


@functools.partial(jax.jit, static_argnames=())
def kernel(x, We1, be1, We2, be2, We3, be3, Wd1, bd1, Wd2, bd2, Wd3, bd3):
    full = lambda s: pl.BlockSpec(s, lambda b: (0,) * len(s))
    z, out = pl.pallas_call(
        _body,
        grid=(_NSTEPS + 1,),
        in_specs=[
            pl.BlockSpec((_BPS * _N, _D),
                         lambda b: (jnp.minimum(b, _NSTEPS - 1), 0)),
            full((_D, _C)), full((1, _C)),
            full((_C, _C)), full((1, _C)),
            full((_C, _C)), full((1, _C)),
            full((_C, _C)), full((1, _C)),
            full((_C, _C)), full((1, _C)),
            full((_C, _M * _D)), full((1, _M * _D)),
        ],
        out_specs=[
            pl.BlockSpec((_B, _C), lambda b: (0, 0)),
            pl.BlockSpec((_B, _M * _D), lambda b: (0, 0)),
        ],
        out_shape=[
            jax.ShapeDtypeStruct((_B, _C), jnp.float32),
            jax.ShapeDtypeStruct((_B, _M * _D), jnp.float32),
        ],
    )(x.reshape(_B * _N, _D), We1, be1.reshape(1, _C), We2, be2.reshape(1, _C),
      We3, be3.reshape(1, _C), Wd1, bd1.reshape(1, _C),
      Wd2, bd2.reshape(1, _C), Wd3, bd3.reshape(1, _M * _D))
    return (out.reshape(_B, _M, _D), z)


# speedup vs baseline: 1.0944x; 1.0535x over previous
import functools
import math

import jax
import jax.numpy as jnp
from jax.experimental import pallas as pl
from jax.experimental.pallas import tpu as pltpu

_B = 16
_N = 2048
_D = 3
_C = 128
_M = 1000


def _gelu(t):
    return t * (0.5 + 0.5 * jax.lax.erf(t * (1.0 / math.sqrt(2.0))))


_BPS = 4
_NSTEPS = _B // _BPS


def _body(x_ref, we1_ref, be1_ref, we2_ref, be2_ref, we3_ref, be3_ref,
          wd1_ref, bd1_ref, wd2_ref, bd2_ref, wd3_ref, bd3_ref,
          z_ref, o_ref, wd3_vmem, wd3_sem):
    b = pl.program_id(0)
    copy = pltpu.make_async_copy(wd3_ref, wd3_vmem, wd3_sem)

    @pl.when(b == 0)
    def _start():
        copy.start()

    @pl.when(b < _NSTEPS)
    def _encode():
        xb = x_ref[...]
        h = (jnp.dot(xb, we1_ref[...], preferred_element_type=jnp.float32)
             + be1_ref[...])
        h = _gelu(h)
        h = _gelu(jnp.dot(h, we2_ref[...], preferred_element_type=jnp.float32)
                  + be2_ref[...])
        a = (jnp.dot(h, we3_ref[...], preferred_element_type=jnp.float32)
             + be3_ref[...])
        a3 = a.reshape(_BPS, _N, _C)
        zrow = jnp.maximum(_gelu(jnp.max(a3, axis=1)),
                           _gelu(jnp.min(a3, axis=1)))
        z_ref[pl.ds(b * _BPS, _BPS), :] = zrow

    @pl.when(b == _NSTEPS)
    def _decode():
        copy.wait()
        d = _gelu(jnp.dot(z_ref[...], wd1_ref[...],
                          preferred_element_type=jnp.float32) + bd1_ref[...])
        d = _gelu(jnp.dot(d, wd2_ref[...],
                          preferred_element_type=jnp.float32) + bd2_ref[...])
        o_ref[...] = (jnp.dot(d, wd3_vmem[...],
                              preferred_element_type=jnp.float32)
                      + bd3_ref[...])


@functools.partial(jax.jit, static_argnames=())
def kernel(x, We1, be1, We2, be2, We3, be3, Wd1, bd1, Wd2, bd2, Wd3, bd3):
    full = lambda s: pl.BlockSpec(s, lambda b: (0,) * len(s))
    z, out = pl.pallas_call(
        _body,
        grid=(_NSTEPS + 1,),
        in_specs=[
            pl.BlockSpec((_BPS * _N, _D),
                         lambda b: (jnp.minimum(b, _NSTEPS - 1), 0)),
            full((_D, _C)), full((1, _C)),
            full((_C, _C)), full((1, _C)),
            full((_C, _C)), full((1, _C)),
            full((_C, _C)), full((1, _C)),
            full((_C, _C)), full((1, _C)),
            pl.BlockSpec(memory_space=pl.ANY), full((1, _M * _D)),
        ],
        scratch_shapes=[
            pltpu.VMEM((_C, _M * _D), jnp.float32),
            pltpu.SemaphoreType.DMA,
        ],
        out_specs=[
            pl.BlockSpec((_B, _C), lambda b: (0, 0)),
            pl.BlockSpec((_B, _M * _D), lambda b: (0, 0)),
        ],
        out_shape=[
            jax.ShapeDtypeStruct((_B, _C), jnp.float32),
            jax.ShapeDtypeStruct((_B, _M * _D), jnp.float32),
        ],
    )(x.reshape(_B * _N, _D), We1, be1.reshape(1, _C), We2, be2.reshape(1, _C),
      We3, be3.reshape(1, _C), Wd1, bd1.reshape(1, _C),
      Wd2, bd2.reshape(1, _C), Wd3, bd3.reshape(1, _M * _D))
    return (out.reshape(_B, _M, _D), z)
